# baseline (speedup 1.0000x reference)
import functools

import jax
import jax.numpy as jnp
from jax import lax
from jax.experimental import pallas as pl
from jax.experimental.pallas import tpu as pltpu
from jax.experimental.pallas import tpu_sc as plsc

_info = plsc.get_sparse_core_info()
_NC = _info.num_cores
_NS = _info.num_subcores
_NW = _NC * _NS

_NBUF = 4
_LP = 56


@functools.partial(jax.jit, static_argnames=("b", "l", "d"))
def _sc_gather(Wp, idx, *, b, l, d):
    n_per_w = _LP * b // _NW
    rows_per_w = b // _NW
    n_groups = rows_per_w // _NBUF
    mesh = plsc.VectorSubcoreMesh(core_axis_name="c", subcore_axis_name="s")

    @functools.partial(
        pl.kernel,
        mesh=mesh,
        out_type=jax.ShapeDtypeStruct((b, l, d), jnp.float32),
        scratch_types=[
            pltpu.VMEM((n_per_w,), jnp.int32),
        ]
        + [pltpu.VMEM((_LP, 2 * d), jnp.float32)] * _NBUF
        + [pltpu.SemaphoreType.DMA] * 2,
        compiler_params=pltpu.CompilerParams(use_tc_tiling_on_sc=True),
    )
    def k(table_hbm, idx_hbm, out_hbm, idx_v, *rest):
        bufs = rest[:_NBUF]
        gsem, osem = rest[_NBUF:]
        wid = lax.axis_index("s") * _NC + lax.axis_index("c")
        base = wid * n_per_w
        row0 = wid * rows_per_w
        pltpu.sync_copy(idx_hbm.at[pl.ds(base, n_per_w)], idx_v)

        def gather(j, s):
            off = pl.multiple_of(j * _LP, _LP)
            return pltpu.make_async_copy(
                table_hbm.at[idx_v.at[pl.ds(off, _LP)]], bufs[s], gsem
            )

        def rowcp(j, s, ll):
            return pltpu.make_async_copy(
                bufs[s].at[ll, pl.ds(0, d)], out_hbm.at[row0 + j, ll], osem
            )

        for s in range(_NBUF):
            gather(s, s).start()

        def body(i, carry):
            jb = i * _NBUF
            for s in range(_NBUF):
                j = jb + s

                @pl.when(i > 0)
                def _drain():
                    for ll in range(l):
                        rowcp(j - _NBUF, s, ll).wait()

                gather(j, s).wait()
                for ll in range(l):
                    rowcp(j, s, ll).start()

                @pl.when(i < n_groups - 1)
                def _refill():
                    gather(j + _NBUF, s).start()

            return carry

        lax.fori_loop(0, n_groups, body, 0)
        for s in range(_NBUF):
            for ll in range(l):
                rowcp((n_groups - 1) * _NBUF + s, s, ll).wait()

    return k(Wp, idx)


def kernel(input_, W):
    b, l = input_.shape
    v, d = W.shape
    idx = jnp.pad(input_, ((0, 0), (0, _LP - l))).reshape(b * _LP)
    Wp = jnp.pad(W, ((0, 0), (0, d)))
    return _sc_gather(Wp, idx, b=b, l=l, d=d)


# speedup vs baseline: 1.0001x; 1.0001x over previous
import functools

import jax
import jax.numpy as jnp
from jax import lax
from jax.experimental import pallas as pl
from jax.experimental.pallas import tpu as pltpu
from jax.experimental.pallas import tpu_sc as plsc

_info = plsc.get_sparse_core_info()
_NC = _info.num_cores
_NS = _info.num_subcores
_NW = _NC * _NS

_NBUF = 4
_LP = 56


@functools.partial(jax.jit, static_argnames=("b", "l", "d"))
def _sc_gather(Wp, idx, *, b, l, d):
    n_per_w = _LP * b // _NW
    rows_per_w = b // _NW
    n_groups = rows_per_w // _NBUF
    mesh = plsc.VectorSubcoreMesh(core_axis_name="c", subcore_axis_name="s")

    @functools.partial(
        pl.kernel,
        mesh=mesh,
        out_type=jax.ShapeDtypeStruct((b, l, d), jnp.float32),
        scratch_types=[
            pltpu.VMEM((n_per_w,), jnp.int32),
        ]
        + [pltpu.VMEM((_LP, 2 * d), jnp.float32)] * _NBUF
        + [pltpu.VMEM((l * d,), jnp.int32)]
        + [pltpu.SemaphoreType.DMA] * 2,
        compiler_params=pltpu.CompilerParams(use_tc_tiling_on_sc=True),
    )
    def k(table_hbm, idx_hbm, out_hbm, idx_v, *rest):
        bufs = rest[:_NBUF]
        drain_v, gsem, osem = rest[_NBUF:]
        wid = lax.axis_index("s") * _NC + lax.axis_index("c")
        base = wid * n_per_w
        row0 = wid * rows_per_w
        pltpu.sync_copy(idx_hbm.at[pl.ds(base, n_per_w)], idx_v)

        def gather(j, s):
            off = pl.multiple_of(j * _LP, _LP)
            return pltpu.make_async_copy(
                table_hbm.at[idx_v.at[pl.ds(off, _LP)]], bufs[s], gsem
            )

        def rowcp(j, s, ll):
            return pltpu.make_async_copy(
                bufs[s].at[ll, pl.ds(0, d)], out_hbm.at[row0 + j, ll], osem
            )

        def slab_drain():
            return pltpu.make_async_copy(
                idx_hbm.at[pl.ds(0, l * d)], drain_v, osem
            )

        for s in range(_NBUF):
            gather(s, s).start()

        def body(i, carry):
            jb = i * _NBUF
            for s in range(_NBUF):
                j = jb + s

                @pl.when(i > 0)
                def _drain():
                    slab_drain().wait()

                gather(j, s).wait()
                for ll in range(l):
                    rowcp(j, s, ll).start()

                @pl.when(i < n_groups - 1)
                def _refill():
                    gather(j + _NBUF, s).start()

            return carry

        lax.fori_loop(0, n_groups, body, 0)
        for s in range(_NBUF):
            slab_drain().wait()

    return k(Wp, idx)


def kernel(input_, W):
    b, l = input_.shape
    v, d = W.shape
    idx = jnp.pad(input_, ((0, 0), (0, _LP - l))).reshape(b * _LP)
    Wp = jnp.pad(W, ((0, 0), (0, d)))
    return _sc_gather(Wp, idx, b=b, l=l, d=d)


# speedup vs baseline: 4.4921x; 4.4914x over previous
import functools

import jax
import jax.numpy as jnp
from jax import lax
from jax.experimental import pallas as pl
from jax.experimental.pallas import tpu as pltpu
from jax.experimental.pallas import tpu_sc as plsc

_info = plsc.get_sparse_core_info()
_NC = _info.num_cores
_NS = _info.num_subcores
_NW = _NC * _NS
_L = _info.num_lanes


@functools.partial(jax.jit, static_argnames=("n", "d", "chunk"))
def _sc_gather(Wp, idx, *, n, d, chunk):
    n_per_w = n // _NW
    n_chunks = n_per_w // chunk
    half = chunk // 2
    mesh = plsc.VectorSubcoreMesh(core_axis_name="c", subcore_axis_name="s")

    @functools.partial(
        pl.kernel,
        mesh=mesh,
        out_type=jax.ShapeDtypeStruct((n * d // 128, 128), jnp.float32),
        scratch_types=[
            pltpu.VMEM((n_per_w,), jnp.int32),
            pltpu.VMEM((chunk, 2 * d), jnp.float32),
            pltpu.VMEM((chunk, 2 * d), jnp.float32),
            pltpu.VMEM((half, 2 * d), jnp.float32),
            pltpu.VMEM((half, 2 * d), jnp.float32),
            pltpu.SemaphoreType.DMA,
            pltpu.SemaphoreType.DMA,
            pltpu.SemaphoreType.DMA,
            pltpu.SemaphoreType.DMA,
        ],
        compiler_params=pltpu.CompilerParams(use_tc_tiling_on_sc=True),
    )
    def k(table_hbm, idx_hbm, out_hbm, idx_v, ga, gb_, pa, pb_, g0, g1, s0, s1):
        wid = lax.axis_index("s") * _NC + lax.axis_index("c")
        base = wid * n_per_w
        pltpu.sync_copy(idx_hbm.at[pl.ds(base, n_per_w)], idx_v)

        gbufs = (ga, gb_)
        pbufs = (pa, pb_)
        gsems = (g0, g1)
        osems = (s0, s1)

        def gather(i, b):
            return pltpu.make_async_copy(
                table_hbm.at[idx_v.at[pl.ds(i * chunk, chunk)]], gbufs[b], gsems[b]
            )

        def outcp(i, b):
            off = pl.multiple_of((base + i * chunk) * d // 128, half)
            return pltpu.make_async_copy(
                pbufs[b], out_hbm.at[pl.ds(off, half)], osems[b]
            )

        def pack(b):
            src = gbufs[b]
            dst = pbufs[b]

            def body(q, carry):
                for kk in range(d // _L):
                    dst[q, pl.ds(kk * _L, _L)] = src[2 * q, pl.ds(kk * _L, _L)]
                    dst[q, pl.ds(d + kk * _L, _L)] = src[
                        2 * q + 1, pl.ds(kk * _L, _L)
                    ]
                return carry

            lax.fori_loop(0, half, body, 0)

        gather(0, 0).start()
        for i in range(n_chunks):
            b = i % 2
            gather(i, b).wait()
            if i + 1 < n_chunks:
                gather(i + 1, (i + 1) % 2).start()
            if i >= 2:
                outcp(i - 2, b).wait()
            pack(b)
            outcp(i, b).start()
        outcp(n_chunks - 2, n_chunks % 2).wait()
        outcp(n_chunks - 1, (n_chunks - 1) % 2).wait()

    return k(Wp, idx)


def kernel(input_, W):
    b, l = input_.shape
    v, d = W.shape
    n = b * l
    idx = input_.reshape(n)
    Wp = jnp.pad(W, ((0, 0), (0, d)))
    out2 = _sc_gather(Wp, idx, n=n, d=d, chunk=256)
    return out2.reshape(b, l, d)


# speedup vs baseline: 4.5058x; 1.0031x over previous
import functools

import jax
import jax.numpy as jnp
from jax import lax
from jax.experimental import pallas as pl
from jax.experimental.pallas import tpu as pltpu
from jax.experimental.pallas import tpu_sc as plsc

_info = plsc.get_sparse_core_info()
_NC = _info.num_cores
_NS = _info.num_subcores
_NW = _NC * _NS
_L = _info.num_lanes


@functools.partial(jax.jit, static_argnames=("v", "n", "d", "chunk"))
def _sc_gather(Wf, idx, *, v, n, d, chunk):
    n_per_w = n // _NW
    n_chunks = n_per_w // chunk
    mesh = plsc.VectorSubcoreMesh(core_axis_name="c", subcore_axis_name="s")

    @functools.partial(
        pl.kernel,
        mesh=mesh,
        out_type=jax.ShapeDtypeStruct((n, d), jnp.float32),
        scratch_types=[
            pltpu.VMEM((v * d,), jnp.float32),
            pltpu.VMEM((n_per_w,), jnp.int32),
            pltpu.VMEM((chunk, d), jnp.float32),
            pltpu.VMEM((chunk, d), jnp.float32),
            pltpu.SemaphoreType.DMA,
            pltpu.SemaphoreType.DMA,
        ],
        compiler_params=pltpu.CompilerParams(use_tc_tiling_on_sc=False),
    )
    def k(table_hbm, idx_hbm, out_hbm, tab_v, idx_v, r0, r1, s0, s1):
        wid = lax.axis_index("s") * _NC + lax.axis_index("c")
        base = wid * n_per_w
        pltpu.sync_copy(table_hbm, tab_v)
        pltpu.sync_copy(idx_hbm.at[pl.ds(base, n_per_w)], idx_v)

        bufs = (r0, r1)
        osems = (s0, s1)

        def outcp(i, b):
            return pltpu.make_async_copy(
                bufs[b], out_hbm.at[pl.ds(base + i * chunk, chunk)], osems[b]
            )

        def fill(c, b):
            buf = bufs[b]

            def body(q16, carry):
                vec = idx_v[pl.ds(c * chunk + q16 * _L, _L)]
                for u in range(_L):
                    s = vec[u]
                    q = q16 * _L + u
                    for kk in range(d // _L):
                        off = pl.multiple_of(s * d + kk * _L, _L)
                        buf[q, pl.ds(kk * _L, _L)] = tab_v[pl.ds(off, _L)]
                return carry

            lax.fori_loop(0, chunk // _L, body, 0)

        for c in range(n_chunks):
            b = c % 2
            if c >= 2:
                outcp(c - 2, b).wait()
            fill(c, b)
            outcp(c, b).start()
        outcp(n_chunks - 2, n_chunks % 2).wait()
        outcp(n_chunks - 1, (n_chunks - 1) % 2).wait()

    return k(Wf, idx)


def kernel(input_, W):
    b, l = input_.shape
    v, d = W.shape
    n = b * l
    idx = input_.reshape(n)
    Wf = W.reshape(v * d)
    out = _sc_gather(Wf, idx, v=v, n=n, d=d, chunk=400)
    return out.reshape(b, l, d)


# speedup vs baseline: 5.0011x; 1.1099x over previous
import functools

import jax
import jax.numpy as jnp
from jax import lax
from jax.experimental import pallas as pl
from jax.experimental.pallas import tpu as pltpu
from jax.experimental.pallas import tpu_sc as plsc

_info = plsc.get_sparse_core_info()
_NC = _info.num_cores
_NS = _info.num_subcores
_NW = _NC * _NS


@functools.partial(jax.jit, static_argnames=("n", "d", "chunk"))
def _sc_gather(W, idx, *, n, d, chunk):
    n_per_w = n // _NW
    n_chunks = n_per_w // chunk
    mesh = plsc.VectorSubcoreMesh(core_axis_name="c", subcore_axis_name="s")

    @functools.partial(
        pl.kernel,
        mesh=mesh,
        out_type=jax.ShapeDtypeStruct((n, d), jnp.float32),
        scratch_types=[
            pltpu.VMEM((n_per_w,), jnp.int32),
            pltpu.VMEM((chunk, d), jnp.float32),
            pltpu.VMEM((chunk, d), jnp.float32),
            pltpu.SemaphoreType.DMA,
            pltpu.SemaphoreType.DMA,
            pltpu.SemaphoreType.DMA,
            pltpu.SemaphoreType.DMA,
        ],
        compiler_params=pltpu.CompilerParams(use_tc_tiling_on_sc=False),
    )
    def k(table_hbm, idx_hbm, out_hbm, idx_v, r0, r1, g0, g1, s0, s1):
        wid = lax.axis_index("s") * _NC + lax.axis_index("c")
        base = wid * n_per_w
        pltpu.sync_copy(idx_hbm.at[pl.ds(base, n_per_w)], idx_v)

        bufs = (r0, r1)
        gsems = (g0, g1)
        osems = (s0, s1)

        def gather(i, b):
            return pltpu.make_async_copy(
                table_hbm.at[idx_v.at[pl.ds(i * chunk, chunk)]], bufs[b], gsems[b]
            )

        def outcp(i, b):
            return pltpu.make_async_copy(
                bufs[b], out_hbm.at[pl.ds(base + i * chunk, chunk)], osems[b]
            )

        gather(0, 0).start()
        for i in range(n_chunks):
            b = i % 2
            nb = (i + 1) % 2
            if i + 1 < n_chunks:
                if i > 0:
                    outcp(i - 1, nb).wait()
                gather(i + 1, nb).start()
            gather(i, b).wait()
            outcp(i, b).start()
        outcp(n_chunks - 2, n_chunks % 2).wait()
        outcp(n_chunks - 1, (n_chunks - 1) % 2).wait()

    return k(W, idx)


def kernel(input_, W):
    b, l = input_.shape
    v, d = W.shape
    n = b * l
    idx = input_.reshape(n)
    out = _sc_gather(W, idx, n=n, d=d, chunk=800)
    return out.reshape(b, l, d)


# speedup vs baseline: 5.2100x; 1.0418x over previous
import functools

import jax
import jax.numpy as jnp
from jax import lax
from jax.experimental import pallas as pl
from jax.experimental.pallas import tpu as pltpu
from jax.experimental.pallas import tpu_sc as plsc

_info = plsc.get_sparse_core_info()
_NC = _info.num_cores
_NS = _info.num_subcores
_NW = _NC * _NS
_L = _info.num_lanes

_CHUNK = 160
_N_STREAM = 23


@functools.partial(jax.jit, static_argnames=("v", "n", "d"))
def _sc_gather(W, idx, *, v, n, d):
    chunk = _CHUNK
    n_per_w = n // _NW
    n_chunks = n_per_w // chunk
    ns = _N_STREAM
    nf = n_chunks - ns
    mesh = plsc.VectorSubcoreMesh(core_axis_name="c", subcore_axis_name="s")

    @functools.partial(
        pl.kernel,
        mesh=mesh,
        out_type=jax.ShapeDtypeStruct((n, d), jnp.float32),
        scratch_types=[
            pltpu.VMEM((v, d), jnp.float32),
            pltpu.VMEM((n_per_w,), jnp.int32),
            pltpu.VMEM((chunk, d), jnp.float32),
            pltpu.VMEM((chunk, d), jnp.float32),
            pltpu.VMEM((chunk, d), jnp.float32),
            pltpu.VMEM((chunk, d), jnp.float32),
            pltpu.SemaphoreType.DMA,
            pltpu.SemaphoreType.DMA,
            pltpu.SemaphoreType.DMA,
            pltpu.SemaphoreType.DMA,
        ],
        compiler_params=pltpu.CompilerParams(use_tc_tiling_on_sc=False),
    )
    def k(table_hbm, idx_hbm, out_hbm, tab_v, idx_v,
          sb0, sb1, fb0, fb1, g0, g1, so, fo):
        wid = lax.axis_index("s") * _NC + lax.axis_index("c")
        base = wid * n_per_w
        pltpu.sync_copy(table_hbm, tab_v)
        pltpu.sync_copy(idx_hbm.at[pl.ds(base, n_per_w)], idx_v)

        sbufs = (sb0, sb1)
        gsems = (g0, g1)
        fbufs = (fb0, fb1)

        def sgather(i, b):
            return pltpu.make_async_copy(
                table_hbm.at[idx_v.at[pl.ds(i * chunk, chunk)]], sbufs[b], gsems[b]
            )

        def outcp(i, buf, sem):
            return pltpu.make_async_copy(
                buf, out_hbm.at[pl.ds(base + i * chunk, chunk)], sem
            )

        def fill(c, b):
            buf = fbufs[b]

            def body(q16, carry):
                vec = idx_v[pl.ds(c * chunk + q16 * _L, _L)]
                for u in range(_L):
                    q = q16 * _L + u
                    s = vec[u]
                    for kk in range(d // _L):
                        buf[q, pl.ds(kk * _L, _L)] = tab_v[s, pl.ds(kk * _L, _L)]
                return carry

            lax.fori_loop(0, chunk // _L, body, 0)

        sgather(0, 0).start()
        for t in range(max(ns, nf)):
            if t < ns:
                b = t % 2
                nb = (t + 1) % 2
                if t + 1 < ns:
                    if t > 0:
                        outcp(t - 1, sbufs[nb], so).wait()
                    sgather(t + 1, nb).start()
                sgather(t, b).wait()
                outcp(t, sbufs[b], so).start()
            if t < nf:
                fb = t % 2
                if t >= 2:
                    outcp(ns + t - 2, fbufs[fb], fo).wait()
                fill(ns + t, fb)
                outcp(ns + t, fbufs[fb], fo).start()
        outcp(ns - 2, sbufs[ns % 2], so).wait()
        outcp(ns - 1, sbufs[(ns - 1) % 2], so).wait()
        outcp(ns + nf - 2, fbufs[nf % 2], fo).wait()
        outcp(ns + nf - 1, fbufs[(nf - 1) % 2], fo).wait()

    return k(W, idx)


def kernel(input_, W):
    b, l = input_.shape
    v, d = W.shape
    n = b * l
    idx = input_.reshape(n)
    out = _sc_gather(W, idx, v=v, n=n, d=d)
    return out.reshape(b, l, d)
